# Initial kernel scaffold; baseline (speedup 1.0000x reference)
import jax
import jax.numpy as jnp
from jax.experimental import pallas as pl


def kernel(predictions, targets):
    raise NotImplementedError("write your pallas kernel here")


# speedup vs baseline: 1.7794x; 1.7794x over previous
import functools

import jax
import jax.numpy as jnp
from jax import lax
from jax.experimental import pallas as pl
from jax.experimental.pallas import tpu as pltpu
from jax.experimental.pallas import tpu_sc as plsc

_K = 4
_LAMBDA_CENTER = 20.0
_EPS = 1e-08

_B = 128
_T = 2048
_L = 16
_NW = 16
_ROWS_PER_W = _B // _NW
_CHUNKS = _T // _L
_LN2 = 0.6931471805599453
_SQRT2 = 1.4142135623730951


def _ln(x):
    bits = plsc.bitcast(x, jnp.int32)
    e = (bits >> 23) - 127
    mb = (bits & 0x007FFFFF) | 0x3F800000
    m = plsc.bitcast(mb, jnp.float32)
    big = m > _SQRT2
    m = jnp.where(big, m * 0.5, m)
    e = jnp.where(big, e + 1, e)
    t = (m - 1.0) / (m + 1.0)
    t2 = t * t
    p = 2.0 * t * (1.0 + t2 * (1.0 / 3.0 + t2 * (1.0 / 5.0 + t2 * (
        1.0 / 7.0 + t2 * (1.0 / 9.0)))))
    return e.astype(jnp.float32) * _LN2 + p


def _insert(ts, v):
    t1, t2, t3, t4 = ts
    n1 = jnp.maximum(t1, v)
    l1 = jnp.minimum(t1, v)
    n2 = jnp.maximum(t2, l1)
    l2 = jnp.minimum(t2, l1)
    n3 = jnp.maximum(t3, l2)
    l3 = jnp.minimum(t3, l2)
    n4 = jnp.maximum(t4, l3)
    return (n1, n2, n3, n4)


def _make_kernel():
    mesh = plsc.VectorSubcoreMesh(
        core_axis_name="c", subcore_axis_name="s", num_cores=1)

    @functools.partial(
        pl.kernel,
        mesh=mesh,
        compiler_params=pltpu.CompilerParams(needs_layout_passes=False),
        out_type=jax.ShapeDtypeStruct((_L,), jnp.float32),
        scratch_types=[
            pltpu.VMEM((_ROWS_PER_W, _T), jnp.float32),
            pltpu.VMEM((_ROWS_PER_W,), jnp.int32),
            pltpu.VMEM_SHARED((_NW * _L,), jnp.float32),
            pltpu.VMEM((_NW * _L,), jnp.float32),
            pltpu.VMEM((_L,), jnp.float32),
            pltpu.VMEM((_L,), jnp.float32),
            pltpu.VMEM((_L,), jnp.float32),
        ],
    )
    def dmil_center_loss(pred_hbm, tgt_hbm, out_hbm,
                         rows_v, tgt_v, sh, all_v, pv, accv, outv):
        wid = lax.axis_index("s")
        base = wid * _ROWS_PER_W
        pltpu.sync_copy(pred_hbm.at[pl.ds(base, _ROWS_PER_W)], rows_v)
        pltpu.sync_copy(tgt_hbm.at[pl.ds(base, _ROWS_PER_W)], tgt_v)

        iota = lax.iota(jnp.int32, _L)
        negv = jnp.full((_L,), -jnp.inf, jnp.float32)
        zerov = jnp.zeros((_L,), jnp.float32)

        mvec_a = zerov
        mvec_b = zerov
        svec = zerov
        qvec = zerov

        for r in range(_ROWS_PER_W):
            def chunk_body(c, carry, r=r):
                g0, g1, s0, s1, q0, q1 = carry
                col = c * (2 * _L)
                v0 = rows_v[r, pl.ds(col, _L)]
                v1 = rows_v[r, pl.ds(col + _L, _L)]
                g0 = _insert(g0, v0)
                g1 = _insert(g1, v1)
                s0 = s0 + v0
                s1 = s1 + v1
                q0 = q0 + v0 * v0
                q1 = q1 + v1 * v1
                return (g0, g1, s0, s1, q0, q1)

            init = ((negv, negv, negv, negv), (negv, negv, negv, negv),
                    zerov, zerov, zerov, zerov)
            g0, g1, s0, s1, q0, q1 = lax.fori_loop(
                0, _CHUNKS // 2, chunk_body, init)

            for v in g1:
                g0 = _insert(g0, v)
            t1, t2, t3, t4 = g0

            s_r = jnp.sum(s0 + s1)
            q_r = jnp.sum(q0 + q1)
            svec = jnp.where(iota == r, s_r, svec)
            qvec = jnp.where(iota == r, q_r, qvec)
            for k in range(_K):
                m = jnp.max(t1)
                l = plsc.all_reduce_ffs(t1 == m)
                sel = iota == l
                t1 = jnp.where(sel, t2, t1)
                t2 = jnp.where(sel, t3, t2)
                t3 = jnp.where(sel, t4, t3)
                t4 = jnp.where(sel, negv, t4)
                lane = 4 * (r % 4) + k
                if r < 4:
                    mvec_a = jnp.where(iota == lane, m, mvec_a)
                else:
                    mvec_b = jnp.where(iota == lane, m, mvec_b)

        row_a = iota >> 2
        y_a = plsc.load_gather(tgt_v, [row_a])
        y_b = plsc.load_gather(tgt_v, [row_a + 4])
        arg_a = jnp.where(y_a == 1, mvec_a + _EPS, 1.0 - mvec_a + _EPS)
        arg_b = jnp.where(y_b == 1, mvec_b + _EPS, 1.0 - mvec_b + _EPS)
        dm_vec = (_ln(arg_a) + _ln(arg_b)) * (-1.0 / _K)
        dmil_p = jnp.sum(dm_vec)

        y8 = plsc.load_gather(tgt_v, [jnp.minimum(iota, _ROWS_PER_W - 1)])
        cc = qvec - svec * svec * (1.0 / _T)
        norm_mask = (iota < _ROWS_PER_W) & (y8 == 0)
        ce_p = jnp.sum(jnp.where(norm_mask, cc, 0.0))
        nn_p = jnp.sum(jnp.where(norm_mask, 1.0, 0.0))

        p = jnp.where(iota == 0, dmil_p,
                      jnp.where(iota == 1, ce_p,
                                jnp.where(iota == 2, nn_p, 0.0)))
        pv[...] = p
        pltpu.sync_copy(pv, sh.at[pl.ds(wid * _L, _L)])
        plsc.subcore_barrier()

        @pl.when(wid == 0)
        def _():
            pltpu.sync_copy(sh, all_v)
            acc = zerov
            for i in range(_NW):
                acc = acc + all_v[pl.ds(i * _L, _L)]
            accv[...] = acc
            zi = jnp.zeros((_L,), jnp.int32)
            dm = plsc.load_gather(accv, [zi])
            ce = plsc.load_gather(accv, [zi + 1])
            nn = plsc.load_gather(accv, [zi + 2])
            dmil = dm * (1.0 / _B)
            denom = jnp.maximum(nn * float(_T), 1.0)
            cl = jnp.where(nn > 0.0, ce / denom, 0.0)
            outv[...] = dmil + _LAMBDA_CENTER * cl
            pltpu.sync_copy(outv, out_hbm)

    return dmil_center_loss


_dmil_sc = _make_kernel()


def kernel(predictions, targets):
    out = _dmil_sc(predictions, targets)
    return out[0]
